# baseline (speedup 1.0000x reference)
import jax
import jax.numpy as jnp
from jax.experimental import pallas as pl
from jax.experimental.pallas import tpu as pltpu

N_RAW = 5000
N = 5120
B = 512
NB = N // B
CONF = 0.25
IOU = 0.45
F32 = jnp.float32
BF16 = jnp.bfloat16


def _iotai(shape, dim):
    return jax.lax.broadcasted_iota(jnp.int32, shape, dim)


def _iota(shape, dim):
    return _iotai(shape, dim).astype(F32)


def _split3(v):
    p1 = v.astype(BF16)
    r = v - p1.astype(F32)
    p2 = r.astype(BF16)
    p3 = (r - p2.astype(F32)).astype(BF16)
    return p1, p2, p3


_DN_T = (((1,), (1,)), ((), ()))


def _t_pieces(eyeb, p1, p2, p3):
    d = lambda p: jax.lax.dot_general(eyeb, p, _DN_T,
                                      preferred_element_type=F32)
    return d(p1) + (d(p2) + d(p3))


def _t_block(eyeb, blk):
    return _t_pieces(eyeb, *_split3(blk))


def _eyesel():
    return _iotai((B, B), 0) == _iotai((B, B), 1)


def _row_to_col(v01):
    return jnp.sum(jnp.where(_eyesel(), v01, 0.0), axis=1, keepdims=True)


def _col_to_row(v01):
    return jnp.sum(jnp.where(_eyesel(), v01, 0.0), axis=0, keepdims=True)


def _iou_gt(bx1, by1, bx2, by2, a1, tx1, ty1, tx2, ty2, a2):
    ix1 = jnp.maximum(bx1, tx1)
    iy1 = jnp.maximum(by1, ty1)
    ix2 = jnp.minimum(bx2, tx2)
    iy2 = jnp.minimum(by2, ty2)
    inter = jnp.clip(ix2 - ix1, 0.0) * jnp.clip(iy2 - iy1, 0.0)
    return (inter / (a1 + a2 - inter + 1e-9)) > IOU


def _nms_kernel(xt_ref, out_ref, dt_ref, st_ref, rkc_ref, krow_ref,
                mu_ref, ml_ref, h1_ref, h2_ref, h3_ref, eye_ref):
    f32 = F32
    eye_ref[...] = jnp.where(_iotai((B, B), 0) == _iotai((B, B), 1),
                             1.0, 0.0).astype(BF16)
    eyeb = eye_ref[...]

    xr = xt_ref[...]
    cxr = xr[0:1, :] * 640.0
    cyr = xr[1:2, :] * 640.0
    wr = xr[2:3, :] * 640.0
    hr = xr[3:4, :] * 640.0
    sr = xr[4:5, :] * xr[5:6, :]
    smr = jnp.where(sr > CONF, sr, -1.0)
    dtv = jnp.concatenate(
        [cxr - wr / 2, cyr - hr / 2, cxr + wr / 2, cyr + hr / 2, smr,
         jnp.zeros((3, N), f32)], axis=0)
    dt_ref[...] = dtv

    h1, h2, h3 = _split3(dtv)
    h1_ref[...] = h1
    h2_ref[...] = h2
    h3_ref[...] = h3

    tri = _iotai((B, 1), 0) > _iotai((1, B), 1)
    for t in range(NB):
        ts = slice(t * B, (t + 1) * B)
        si = _t_pieces(eyeb, h1_ref[:, ts], h2_ref[:, ts],
                       h3_ref[:, ts])[:, 4:5]
        acc = jnp.zeros((B, 1), f32)
        for j in range(NB):
            sj = dt_ref[4:5, j * B:(j + 1) * B]
            if j < t:
                hit = sj >= si
            elif j > t:
                hit = sj > si
            else:
                hit = (sj > si) | ((sj == si) & tri)
            acc = acc + jnp.sum(hit.astype(f32), axis=1, keepdims=True)
        rkc_ref[:, t:t + 1] = acc

    dn = (((1,), (0,)), ((), ()))
    nvalid = jnp.sum((smr > CONF).astype(f32))
    for t in range(NB):
        ts = slice(t * B, (t + 1) * B)

        @pl.when(nvalid > float(t * B))
        def _gather(ts=ts, t=t):
            kkr = _iota((1, B), 1) + t * B
            acc = jnp.zeros((8, B), f32)
            for j in range(NB):
                js = slice(j * B, (j + 1) * B)
                q = (rkc_ref[:, j:j + 1] == kkr).astype(BF16)
                m1 = jax.lax.dot_general(
                    h1_ref[:, js], q, dn, preferred_element_type=f32)
                m2 = jax.lax.dot_general(
                    h2_ref[:, js], q, dn, preferred_element_type=f32)
                m3 = jax.lax.dot_general(
                    h3_ref[:, js], q, dn, preferred_element_type=f32)
                acc = acc + (m1 + (m2 + m3))
            st_ref[:, ts] = acc

        @pl.when(nvalid <= float(t * B))
        def _zero(ts=ts):
            st_ref[:, ts] = jnp.zeros((8, B), f32)

    krow_ref[...] = (st_ref[4:5, :] > CONF).astype(f32)
    ar = (jnp.clip(st_ref[2:3, :] - st_ref[0:1, :], 0.0) *
          jnp.clip(st_ref[3:4, :] - st_ref[1:2, :], 0.0))

    one = jnp.ones((B, B), f32)
    zero = jnp.zeros((B, B), f32)
    ri = _iotai((B, B), 0)
    ci = _iotai((B, B), 1)

    for b in range(NB):
        bs = slice(b * B, (b + 1) * B)
        v_row = krow_ref[0:1, bs]

        @pl.when(jnp.sum(v_row) > 0.0)
        def _process(b=b, bs=bs, v_row=v_row):
            tx1 = st_ref[0:1, bs]
            ty1 = st_ref[1:2, bs]
            tx2 = st_ref[2:3, bs]
            ty2 = st_ref[3:4, bs]
            tb = _t_block(eyeb, st_ref[:, bs])
            bx1 = tb[:, 0:1]
            by1 = tb[:, 1:2]
            bx2 = tb[:, 2:3]
            by2 = tb[:, 3:4]
            a1 = (jnp.clip(bx2 - bx1, 0.0) * jnp.clip(by2 - by1, 0.0))
            gt = _iou_gt(bx1, by1, bx2, by2, a1,
                         tx1, ty1, tx2, ty2, ar[0:1, bs])
            mu_ref[...] = jnp.where(gt & (ri < ci), one, zero)
            ml_ref[...] = jnp.where(gt & (ri > ci), one, zero)
            v_col = _row_to_col(v_row)

            def cond(st):
                _, t, diff = st
                return (diff > 0.0) & (t < B + 2)

            def body(st):
                kc, t, _ = st
                sup_r = jnp.max(mu_ref[...] * kc, axis=0, keepdims=True)
                kr = v_row * (1.0 - sup_r)
                sup_c = jnp.max(ml_ref[...] * kr, axis=1, keepdims=True)
                kc2 = v_col * (1.0 - sup_c)
                return kc2, t + 1, jnp.sum(jnp.abs(kc2 - kc))

            kc_fin, _, _ = jax.lax.while_loop(
                cond, body, (v_col, jnp.int32(0), jnp.float32(1.0)))
            krow_ref[0:1, bs] = _col_to_row(kc_fin)

            @pl.when(jnp.sum(kc_fin) > 0.0)
            def _cross():
                kcb = jnp.broadcast_to(kc_fin, (B, B))
                for c in range(b + 1, NB):
                    cs = slice(c * B, (c + 1) * B)
                    ka = krow_ref[0:1, cs]

                    @pl.when(jnp.sum(ka) > 0.0)
                    def _one(c=c, cs=cs, ka=ka):
                        gtc = _iou_gt(bx1, by1, bx2, by2, a1,
                                      st_ref[0:1, cs], st_ref[1:2, cs],
                                      st_ref[2:3, cs], st_ref[3:4, cs],
                                      ar[0:1, cs])
                        sup = jnp.max(jnp.where(gtc, kcb, zero),
                                      axis=0, keepdims=True)
                        krow_ref[0:1, cs] = ka * (1.0 - sup)

    out_ref[...] = st_ref[...] * krow_ref[...]


@jax.jit
def kernel(x):
    p = x[0]
    xt = jnp.zeros((8, N), F32).at[:6, :N_RAW].set(p.T)
    out_t = pl.pallas_call(
        _nms_kernel,
        out_shape=jax.ShapeDtypeStruct((8, N), F32),
        scratch_shapes=[
            pltpu.VMEM((8, N), F32),
            pltpu.VMEM((8, N), F32),
            pltpu.VMEM((B, 16), F32),
            pltpu.VMEM((1, N), F32),
            pltpu.VMEM((B, B), F32),
            pltpu.VMEM((B, B), F32),
            pltpu.VMEM((8, N), BF16),
            pltpu.VMEM((8, N), BF16),
            pltpu.VMEM((8, N), BF16),
            pltpu.VMEM((B, B), BF16),
        ],
    )(xt)
    return out_t[:5, :N_RAW].T


# speedup vs baseline: 1.0160x; 1.0160x over previous
import jax
import jax.numpy as jnp
from jax.experimental import pallas as pl
from jax.experimental.pallas import tpu as pltpu

N_RAW = 5000
N = 5120
B = 512
NB = N // B
CONF = 0.25
IOU = 0.45
F32 = jnp.float32
BF16 = jnp.bfloat16


def _iotai(shape, dim):
    return jax.lax.broadcasted_iota(jnp.int32, shape, dim)


def _iota(shape, dim):
    return _iotai(shape, dim).astype(F32)


def _split3(v):
    p1 = v.astype(BF16)
    r = v - p1.astype(F32)
    p2 = r.astype(BF16)
    p3 = (r - p2.astype(F32)).astype(BF16)
    return p1, p2, p3


_DN_T = (((1,), (1,)), ((), ()))


def _t_pieces(eyeb, p1, p2, p3):
    d = lambda p: jax.lax.dot_general(eyeb, p, _DN_T,
                                      preferred_element_type=F32)
    return d(p1) + (d(p2) + d(p3))


def _t_block(eyeb, blk):
    return _t_pieces(eyeb, *_split3(blk))


def _eyesel():
    return _iotai((B, B), 0) == _iotai((B, B), 1)


def _row_to_col(v01):
    return jnp.sum(jnp.where(_eyesel(), v01, 0.0), axis=1, keepdims=True)


def _col_to_row(v01):
    return jnp.sum(jnp.where(_eyesel(), v01, 0.0), axis=0, keepdims=True)


def _iou_gt(bx1, by1, bx2, by2, a1, tx1, ty1, tx2, ty2, a2):
    ix1 = jnp.maximum(bx1, tx1)
    iy1 = jnp.maximum(by1, ty1)
    ix2 = jnp.minimum(bx2, tx2)
    iy2 = jnp.minimum(by2, ty2)
    inter = jnp.clip(ix2 - ix1, 0.0) * jnp.clip(iy2 - iy1, 0.0)
    return (inter / (a1 + a2 - inter + 1e-9)) > IOU


def _nms_kernel(xt_ref, out_ref, dt_ref, st_ref, rkc_ref, krow_ref,
                mu_ref, ml_ref, h1_ref, h2_ref, h3_ref, eye_ref):
    f32 = F32
    eye_ref[...] = jnp.where(_iotai((B, B), 0) == _iotai((B, B), 1),
                             1.0, 0.0).astype(BF16)
    eyeb = eye_ref[...]

    xr = xt_ref[...]
    cxr = xr[0:1, :] * 640.0
    cyr = xr[1:2, :] * 640.0
    wr = xr[2:3, :] * 640.0
    hr = xr[3:4, :] * 640.0
    sr = xr[4:5, :] * xr[5:6, :]
    smr = jnp.where(sr > CONF, sr, -1.0)
    dtv = jnp.concatenate(
        [cxr - wr / 2, cyr - hr / 2, cxr + wr / 2, cyr + hr / 2, smr,
         jnp.zeros((3, N), f32)], axis=0)
    dt_ref[...] = dtv

    h1, h2, h3 = _split3(dtv)
    h1_ref[...] = h1
    h2_ref[...] = h2
    h3_ref[...] = h3

    tri = _iotai((B, 1), 0) > _iotai((1, B), 1)
    for t in range(NB):
        ts = slice(t * B, (t + 1) * B)
        si = _row_to_col(dt_ref[4:5, ts])
        acc = jnp.zeros((B, 1), f32)
        for j in range(NB):
            sj = dt_ref[4:5, j * B:(j + 1) * B]
            if j < t:
                hit = sj >= si
            elif j > t:
                hit = sj > si
            else:
                hit = (sj > si) | ((sj == si) & tri)
            acc = acc + jnp.sum(hit.astype(f32), axis=1, keepdims=True)
        rkc_ref[:, t:t + 1] = acc

    dn = (((1,), (0,)), ((), ()))
    nvalid = jnp.sum((smr > CONF).astype(f32))
    for t in range(NB):
        ts = slice(t * B, (t + 1) * B)

        @pl.when(nvalid > float(t * B))
        def _gather(ts=ts, t=t):
            kkr = _iota((1, B), 1) + t * B
            acc = jnp.zeros((8, B), f32)
            for j in range(NB):
                js = slice(j * B, (j + 1) * B)
                q = (rkc_ref[:, j:j + 1] == kkr).astype(BF16)
                m1 = jax.lax.dot_general(
                    h1_ref[:, js], q, dn, preferred_element_type=f32)
                m2 = jax.lax.dot_general(
                    h2_ref[:, js], q, dn, preferred_element_type=f32)
                m3 = jax.lax.dot_general(
                    h3_ref[:, js], q, dn, preferred_element_type=f32)
                acc = acc + (m1 + (m2 + m3))
            st_ref[:, ts] = acc

        @pl.when(nvalid <= float(t * B))
        def _zero(ts=ts):
            st_ref[:, ts] = jnp.zeros((8, B), f32)

    krow_ref[...] = (st_ref[4:5, :] > CONF).astype(f32)
    ar = (jnp.clip(st_ref[2:3, :] - st_ref[0:1, :], 0.0) *
          jnp.clip(st_ref[3:4, :] - st_ref[1:2, :], 0.0))

    one = jnp.ones((B, B), f32)
    zero = jnp.zeros((B, B), f32)
    ri = _iotai((B, B), 0)
    ci = _iotai((B, B), 1)

    for b in range(NB):
        bs = slice(b * B, (b + 1) * B)
        v_row = krow_ref[0:1, bs]

        @pl.when(jnp.sum(v_row) > 0.0)
        def _process(b=b, bs=bs, v_row=v_row):
            tx1 = st_ref[0:1, bs]
            ty1 = st_ref[1:2, bs]
            tx2 = st_ref[2:3, bs]
            ty2 = st_ref[3:4, bs]
            tb = _t_block(eyeb, st_ref[:, bs])
            bx1 = tb[:, 0:1]
            by1 = tb[:, 1:2]
            bx2 = tb[:, 2:3]
            by2 = tb[:, 3:4]
            a1 = (jnp.clip(bx2 - bx1, 0.0) * jnp.clip(by2 - by1, 0.0))
            gt = _iou_gt(bx1, by1, bx2, by2, a1,
                         tx1, ty1, tx2, ty2, ar[0:1, bs])
            mu_ref[...] = jnp.where(gt & (ri < ci), one, zero)
            ml_ref[...] = jnp.where(gt & (ri > ci), one, zero)
            v_col = _row_to_col(v_row)

            def cond(st):
                _, t, diff = st
                return (diff > 0.0) & (t < B + 2)

            def body(st):
                kc, t, _ = st
                sup_r = jnp.max(mu_ref[...] * kc, axis=0, keepdims=True)
                kr = v_row * (1.0 - sup_r)
                sup_c = jnp.max(ml_ref[...] * kr, axis=1, keepdims=True)
                kc2 = v_col * (1.0 - sup_c)
                return kc2, t + 1, jnp.sum(jnp.abs(kc2 - kc))

            kc_fin, _, _ = jax.lax.while_loop(
                cond, body, (v_col, jnp.int32(0), jnp.float32(1.0)))
            krow_ref[0:1, bs] = _col_to_row(kc_fin)

            @pl.when(jnp.sum(kc_fin) > 0.0)
            def _cross():
                kcb = jnp.broadcast_to(kc_fin, (B, B))
                for c in range(b + 1, NB):
                    cs = slice(c * B, (c + 1) * B)
                    ka = krow_ref[0:1, cs]

                    @pl.when(jnp.sum(ka) > 0.0)
                    def _one(c=c, cs=cs, ka=ka):
                        gtc = _iou_gt(bx1, by1, bx2, by2, a1,
                                      st_ref[0:1, cs], st_ref[1:2, cs],
                                      st_ref[2:3, cs], st_ref[3:4, cs],
                                      ar[0:1, cs])
                        sup = jnp.max(jnp.where(gtc, kcb, zero),
                                      axis=0, keepdims=True)
                        krow_ref[0:1, cs] = ka * (1.0 - sup)

    out_ref[...] = st_ref[...] * krow_ref[...]


@jax.jit
def kernel(x):
    p = x[0]
    xt = jnp.zeros((8, N), F32).at[:6, :N_RAW].set(p.T)
    out_t = pl.pallas_call(
        _nms_kernel,
        out_shape=jax.ShapeDtypeStruct((8, N), F32),
        scratch_shapes=[
            pltpu.VMEM((8, N), F32),
            pltpu.VMEM((8, N), F32),
            pltpu.VMEM((B, 16), F32),
            pltpu.VMEM((1, N), F32),
            pltpu.VMEM((B, B), F32),
            pltpu.VMEM((B, B), F32),
            pltpu.VMEM((8, N), BF16),
            pltpu.VMEM((8, N), BF16),
            pltpu.VMEM((8, N), BF16),
            pltpu.VMEM((B, B), BF16),
        ],
    )(xt)
    return out_t[:5, :N_RAW].T


# speedup vs baseline: 1.0636x; 1.0468x over previous
import jax
import jax.numpy as jnp
from jax.experimental import pallas as pl
from jax.experimental.pallas import tpu as pltpu

N_RAW = 5000
N = 5120
B = 512
NB = N // B
CONF = 0.25
IOU = 0.45
F32 = jnp.float32
BF16 = jnp.bfloat16


def _iotai(shape, dim):
    return jax.lax.broadcasted_iota(jnp.int32, shape, dim)


def _iota(shape, dim):
    return _iotai(shape, dim).astype(F32)


def _split3(v):
    p1 = v.astype(BF16)
    r = v - p1.astype(F32)
    p2 = r.astype(BF16)
    p3 = (r - p2.astype(F32)).astype(BF16)
    return p1, p2, p3


_DN_T = (((1,), (1,)), ((), ()))


def _t_pieces(eyeb, p1, p2, p3):
    d = lambda p: jax.lax.dot_general(eyeb, p, _DN_T,
                                      preferred_element_type=F32)
    return d(p1) + (d(p2) + d(p3))


def _t_block(eyeb, blk):
    return _t_pieces(eyeb, *_split3(blk))


def _eyesel():
    return _iotai((B, B), 0) == _iotai((B, B), 1)


def _row_to_col(v01):
    return jnp.sum(jnp.where(_eyesel(), v01, 0.0), axis=1, keepdims=True)


def _col_to_row(v01):
    return jnp.sum(jnp.where(_eyesel(), v01, 0.0), axis=0, keepdims=True)


def _iou_gt(bx1, by1, bx2, by2, a1, tx1, ty1, tx2, ty2, a2):
    ix1 = jnp.maximum(bx1, tx1)
    iy1 = jnp.maximum(by1, ty1)
    ix2 = jnp.minimum(bx2, tx2)
    iy2 = jnp.minimum(by2, ty2)
    inter = jnp.clip(ix2 - ix1, 0.0) * jnp.clip(iy2 - iy1, 0.0)
    return (inter / (a1 + a2 - inter + 1e-9)) > IOU


def _nms_kernel(xt_ref, out_ref, dt_ref, st_ref, rkc_ref, krow_ref,
                mu_ref, ml_ref, h1_ref, h2_ref, h3_ref, eye_ref):
    f32 = F32
    eye_ref[...] = jnp.where(_iotai((B, B), 0) == _iotai((B, B), 1),
                             1.0, 0.0).astype(BF16)
    eyeb = eye_ref[...]

    xr = xt_ref[...]
    cxr = xr[0:1, :] * 640.0
    cyr = xr[1:2, :] * 640.0
    wr = xr[2:3, :] * 640.0
    hr = xr[3:4, :] * 640.0
    sr = xr[4:5, :] * xr[5:6, :]
    smr = jnp.where(sr > CONF, sr, -1.0)
    dtv = jnp.concatenate(
        [cxr - wr / 2, cyr - hr / 2, cxr + wr / 2, cyr + hr / 2, smr,
         jnp.zeros((3, N), f32)], axis=0)
    dt_ref[...] = dtv

    h1, h2, h3 = _split3(dtv)
    h1_ref[...] = h1
    h2_ref[...] = h2
    h3_ref[...] = h3

    tri = _iotai((B, 1), 0) > _iotai((1, B), 1)
    acc_rows = [jnp.zeros((1, B), f32) for _ in range(NB)]
    for t in range(NB):
        ts = slice(t * B, (t + 1) * B)
        sit = dt_ref[4:5, ts]
        si = _row_to_col(sit)
        diag = (sit > si) | ((sit == si) & tri)
        acc = jnp.sum(diag.astype(f32), axis=1, keepdims=True)
        for j in range(t + 1, NB):
            sj = dt_ref[4:5, j * B:(j + 1) * B]
            a = (sj > si).astype(f32)
            acc = acc + jnp.sum(a, axis=1, keepdims=True)
            acc_rows[j] = acc_rows[j] + (
                float(B) - jnp.sum(a, axis=0, keepdims=True))
        rkc_ref[:, t:t + 1] = acc + _row_to_col(acc_rows[t])

    dn = (((1,), (0,)), ((), ()))
    nvalid = jnp.sum((smr > CONF).astype(f32))
    for t in range(NB):
        ts = slice(t * B, (t + 1) * B)

        @pl.when(nvalid > float(t * B))
        def _gather(ts=ts, t=t):
            kkr = _iota((1, B), 1) + t * B
            acc = jnp.zeros((8, B), f32)
            for j in range(NB):
                js = slice(j * B, (j + 1) * B)
                q = (rkc_ref[:, j:j + 1] == kkr).astype(BF16)
                m1 = jax.lax.dot_general(
                    h1_ref[:, js], q, dn, preferred_element_type=f32)
                m2 = jax.lax.dot_general(
                    h2_ref[:, js], q, dn, preferred_element_type=f32)
                m3 = jax.lax.dot_general(
                    h3_ref[:, js], q, dn, preferred_element_type=f32)
                acc = acc + (m1 + (m2 + m3))
            st_ref[:, ts] = acc

        @pl.when(nvalid <= float(t * B))
        def _zero(ts=ts):
            st_ref[:, ts] = jnp.zeros((8, B), f32)

    krow_ref[...] = (st_ref[4:5, :] > CONF).astype(f32)
    ar = (jnp.clip(st_ref[2:3, :] - st_ref[0:1, :], 0.0) *
          jnp.clip(st_ref[3:4, :] - st_ref[1:2, :], 0.0))

    one = jnp.ones((B, B), f32)
    zero = jnp.zeros((B, B), f32)
    ri = _iotai((B, B), 0)
    ci = _iotai((B, B), 1)

    for b in range(NB):
        bs = slice(b * B, (b + 1) * B)
        v_row = krow_ref[0:1, bs]

        @pl.when(jnp.sum(v_row) > 0.0)
        def _process(b=b, bs=bs, v_row=v_row):
            tx1 = st_ref[0:1, bs]
            ty1 = st_ref[1:2, bs]
            tx2 = st_ref[2:3, bs]
            ty2 = st_ref[3:4, bs]
            tb = _t_block(eyeb, st_ref[:, bs])
            bx1 = tb[:, 0:1]
            by1 = tb[:, 1:2]
            bx2 = tb[:, 2:3]
            by2 = tb[:, 3:4]
            a1 = (jnp.clip(bx2 - bx1, 0.0) * jnp.clip(by2 - by1, 0.0))
            gt = _iou_gt(bx1, by1, bx2, by2, a1,
                         tx1, ty1, tx2, ty2, ar[0:1, bs])
            mu_ref[...] = jnp.where(gt & (ri < ci), one, zero)
            ml_ref[...] = jnp.where(gt & (ri > ci), one, zero)
            v_col = _row_to_col(v_row)

            def cond(st):
                _, t, diff = st
                return (diff > 0.0) & (t < B + 2)

            def body(st):
                kc, t, _ = st
                sup_r = jnp.max(mu_ref[...] * kc, axis=0, keepdims=True)
                kr = v_row * (1.0 - sup_r)
                sup_c = jnp.max(ml_ref[...] * kr, axis=1, keepdims=True)
                kc2 = v_col * (1.0 - sup_c)
                return kc2, t + 1, jnp.sum(jnp.abs(kc2 - kc))

            kc_fin, _, _ = jax.lax.while_loop(
                cond, body, (v_col, jnp.int32(0), jnp.float32(1.0)))
            krow_ref[0:1, bs] = _col_to_row(kc_fin)

            @pl.when(jnp.sum(kc_fin) > 0.0)
            def _cross():
                kcb = jnp.broadcast_to(kc_fin, (B, B))
                for c in range(b + 1, NB):
                    cs = slice(c * B, (c + 1) * B)
                    ka = krow_ref[0:1, cs]

                    @pl.when(jnp.sum(ka) > 0.0)
                    def _one(c=c, cs=cs, ka=ka):
                        gtc = _iou_gt(bx1, by1, bx2, by2, a1,
                                      st_ref[0:1, cs], st_ref[1:2, cs],
                                      st_ref[2:3, cs], st_ref[3:4, cs],
                                      ar[0:1, cs])
                        sup = jnp.max(jnp.where(gtc, kcb, zero),
                                      axis=0, keepdims=True)
                        krow_ref[0:1, cs] = ka * (1.0 - sup)

    out_ref[...] = st_ref[...] * krow_ref[...]


@jax.jit
def kernel(x):
    p = x[0]
    xt = jnp.zeros((8, N), F32).at[:6, :N_RAW].set(p.T)
    out_t = pl.pallas_call(
        _nms_kernel,
        out_shape=jax.ShapeDtypeStruct((8, N), F32),
        scratch_shapes=[
            pltpu.VMEM((8, N), F32),
            pltpu.VMEM((8, N), F32),
            pltpu.VMEM((B, 16), F32),
            pltpu.VMEM((1, N), F32),
            pltpu.VMEM((B, B), F32),
            pltpu.VMEM((B, B), F32),
            pltpu.VMEM((8, N), BF16),
            pltpu.VMEM((8, N), BF16),
            pltpu.VMEM((8, N), BF16),
            pltpu.VMEM((B, B), BF16),
        ],
    )(xt)
    return out_t[:5, :N_RAW].T


# speedup vs baseline: 1.2460x; 1.1715x over previous
import jax
import jax.numpy as jnp
from jax.experimental import pallas as pl
from jax.experimental.pallas import tpu as pltpu

N_RAW = 5000
N = 5120
B = 512
NB = N // B
CONF = 0.25
IOU = 0.45
F32 = jnp.float32
BF16 = jnp.bfloat16


def _iotai(shape, dim):
    return jax.lax.broadcasted_iota(jnp.int32, shape, dim)


def _iota(shape, dim):
    return _iotai(shape, dim).astype(F32)


def _split3(v):
    p1 = v.astype(BF16)
    r = v - p1.astype(F32)
    p2 = r.astype(BF16)
    p3 = (r - p2.astype(F32)).astype(BF16)
    return p1, p2, p3


_DN_T = (((1,), (1,)), ((), ()))


def _t_pieces(eyeb, p1, p2, p3):
    d = lambda p: jax.lax.dot_general(eyeb, p, _DN_T,
                                      preferred_element_type=F32)
    return d(p1) + (d(p2) + d(p3))


def _t_block(eyeb, blk):
    return _t_pieces(eyeb, *_split3(blk))


def _eyesel():
    return _iotai((B, B), 0) == _iotai((B, B), 1)


def _row_to_col(v01):
    return jnp.sum(jnp.where(_eyesel(), v01, 0.0), axis=1, keepdims=True)


def _col_to_row(v01):
    return jnp.sum(jnp.where(_eyesel(), v01, 0.0), axis=0, keepdims=True)


def _iou_gt(bx1, by1, bx2, by2, a1, tx1, ty1, tx2, ty2, a2):
    ix1 = jnp.maximum(bx1, tx1)
    iy1 = jnp.maximum(by1, ty1)
    ix2 = jnp.minimum(bx2, tx2)
    iy2 = jnp.minimum(by2, ty2)
    inter = jnp.clip(ix2 - ix1, 0.0) * jnp.clip(iy2 - iy1, 0.0)
    return (inter / (a1 + a2 - inter + 1e-9)) > IOU


def _nms_kernel(xt_ref, out_ref, dt_ref, st_ref, rkc_ref, krow_ref,
                mu_ref, ml_ref, hs_ref, eye_ref):
    f32 = F32
    eye_ref[...] = jnp.where(_iotai((B, B), 0) == _iotai((B, B), 1),
                             1.0, 0.0).astype(BF16)
    eyeb = eye_ref[...]

    xr = xt_ref[...]
    cxr = xr[0:1, :] * 640.0
    cyr = xr[1:2, :] * 640.0
    wr = xr[2:3, :] * 640.0
    hr = xr[3:4, :] * 640.0
    sr = xr[4:5, :] * xr[5:6, :]
    smr = jnp.where(sr > CONF, sr, -1.0)
    dtv = jnp.concatenate(
        [cxr - wr / 2, cyr - hr / 2, cxr + wr / 2, cyr + hr / 2, smr,
         jnp.zeros((3, N), f32)], axis=0)
    dt_ref[...] = dtv

    h1, h2, h3 = _split3(dtv)
    hs_ref[...] = jnp.concatenate([h1, h2, h3], axis=0)

    tri = _iotai((B, 1), 0) > _iotai((1, B), 1)
    acc_rows = [jnp.zeros((1, B), f32) for _ in range(NB)]
    for t in range(NB):
        ts = slice(t * B, (t + 1) * B)
        sit = dt_ref[4:5, ts]
        si = _row_to_col(sit)
        diag = (sit > si) | ((sit == si) & tri)
        acc = jnp.sum(diag.astype(f32), axis=1, keepdims=True)
        for j in range(t + 1, NB):
            sj = dt_ref[4:5, j * B:(j + 1) * B]
            a = (sj > si).astype(f32)
            acc = acc + jnp.sum(a, axis=1, keepdims=True)
            acc_rows[j] = acc_rows[j] + (
                float(B) - jnp.sum(a, axis=0, keepdims=True))
        rkc_ref[:, t:t + 1] = acc + _row_to_col(acc_rows[t])

    dn = (((1,), (0,)), ((), ()))
    nvalid = jnp.sum((smr > CONF).astype(f32))
    for t in range(NB):
        ts = slice(t * B, (t + 1) * B)

        @pl.when(nvalid > float(t * B))
        def _gather(ts=ts, t=t):
            kkr = _iota((1, B), 1) + t * B
            acc = jnp.zeros((24, B), f32)
            for j in range(NB):
                js = slice(j * B, (j + 1) * B)
                q = (rkc_ref[:, j:j + 1] == kkr).astype(BF16)
                acc = acc + jax.lax.dot_general(
                    hs_ref[:, js], q, dn, preferred_element_type=f32)
            st_ref[:, ts] = acc[0:8] + (acc[8:16] + acc[16:24])

        @pl.when(nvalid <= float(t * B))
        def _zero(ts=ts):
            st_ref[:, ts] = jnp.zeros((8, B), f32)

    krow_ref[...] = (st_ref[4:5, :] > CONF).astype(f32)
    ar = (jnp.clip(st_ref[2:3, :] - st_ref[0:1, :], 0.0) *
          jnp.clip(st_ref[3:4, :] - st_ref[1:2, :], 0.0))

    one = jnp.ones((B, B), f32)
    zero = jnp.zeros((B, B), f32)
    ri = _iotai((B, B), 0)
    ci = _iotai((B, B), 1)

    for b in range(NB):
        bs = slice(b * B, (b + 1) * B)
        v_row = krow_ref[0:1, bs]

        @pl.when(jnp.sum(v_row) > 0.0)
        def _process(b=b, bs=bs, v_row=v_row):
            tx1 = st_ref[0:1, bs]
            ty1 = st_ref[1:2, bs]
            tx2 = st_ref[2:3, bs]
            ty2 = st_ref[3:4, bs]
            tb = _t_block(eyeb, st_ref[:, bs])
            bx1 = tb[:, 0:1]
            by1 = tb[:, 1:2]
            bx2 = tb[:, 2:3]
            by2 = tb[:, 3:4]
            a1 = (jnp.clip(bx2 - bx1, 0.0) * jnp.clip(by2 - by1, 0.0))
            gt = _iou_gt(bx1, by1, bx2, by2, a1,
                         tx1, ty1, tx2, ty2, ar[0:1, bs])
            mu_ref[...] = jnp.where(gt & (ri < ci), one, zero)
            ml_ref[...] = jnp.where(gt & (ri > ci), one, zero)
            v_col = _row_to_col(v_row)

            def cond(st):
                _, t, diff = st
                return (diff > 0.0) & (t < B + 2)

            def body(st):
                kc, t, _ = st
                sup_r = jnp.max(mu_ref[...] * kc, axis=0, keepdims=True)
                kr = v_row * (1.0 - sup_r)
                sup_c = jnp.max(ml_ref[...] * kr, axis=1, keepdims=True)
                kc2 = v_col * (1.0 - sup_c)
                return kc2, t + 1, jnp.sum(jnp.abs(kc2 - kc))

            kc_fin, _, _ = jax.lax.while_loop(
                cond, body, (v_col, jnp.int32(0), jnp.float32(1.0)))
            krow_ref[0:1, bs] = _col_to_row(kc_fin)

            @pl.when(jnp.sum(kc_fin) > 0.0)
            def _cross():
                kcb = jnp.broadcast_to(kc_fin, (B, B))
                for c in range(b + 1, NB):
                    cs = slice(c * B, (c + 1) * B)
                    ka = krow_ref[0:1, cs]

                    @pl.when(jnp.sum(ka) > 0.0)
                    def _one(c=c, cs=cs, ka=ka):
                        gtc = _iou_gt(bx1, by1, bx2, by2, a1,
                                      st_ref[0:1, cs], st_ref[1:2, cs],
                                      st_ref[2:3, cs], st_ref[3:4, cs],
                                      ar[0:1, cs])
                        sup = jnp.max(jnp.where(gtc, kcb, zero),
                                      axis=0, keepdims=True)
                        krow_ref[0:1, cs] = ka * (1.0 - sup)

    out_ref[...] = st_ref[...] * krow_ref[...]


@jax.jit
def kernel(x):
    p = x[0]
    xt = jnp.zeros((8, N), F32).at[:6, :N_RAW].set(p.T)
    out_t = pl.pallas_call(
        _nms_kernel,
        out_shape=jax.ShapeDtypeStruct((8, N), F32),
        scratch_shapes=[
            pltpu.VMEM((8, N), F32),
            pltpu.VMEM((8, N), F32),
            pltpu.VMEM((B, 16), F32),
            pltpu.VMEM((1, N), F32),
            pltpu.VMEM((B, B), F32),
            pltpu.VMEM((B, B), F32),
            pltpu.VMEM((24, N), BF16),
            pltpu.VMEM((B, B), BF16),
        ],
    )(xt)
    return out_t[:5, :N_RAW].T


# speedup vs baseline: 1.6291x; 1.3075x over previous
import jax
import jax.numpy as jnp
from jax.experimental import pallas as pl
from jax.experimental.pallas import tpu as pltpu

N_RAW = 5000
N = 5120
B = 512
NB = N // B
CONF = 0.25
IOU = 0.45
F32 = jnp.float32
BF16 = jnp.bfloat16


def _iotai(shape, dim):
    return jax.lax.broadcasted_iota(jnp.int32, shape, dim)


def _iota(shape, dim):
    return _iotai(shape, dim).astype(F32)


def _split3(v):
    p1 = v.astype(BF16)
    r = v - p1.astype(F32)
    p2 = r.astype(BF16)
    p3 = (r - p2.astype(F32)).astype(BF16)
    return p1, p2, p3


_DN_T = (((1,), (1,)), ((), ()))


def _t_pieces(eyeb, p1, p2, p3):
    d = lambda p: jax.lax.dot_general(eyeb, p, _DN_T,
                                      preferred_element_type=F32)
    return d(p1) + (d(p2) + d(p3))


def _t_block(eyeb, blk):
    return _t_pieces(eyeb, *_split3(blk))


def _eyesel():
    return _iotai((B, B), 0) == _iotai((B, B), 1)


def _row_to_col(v01):
    return jnp.sum(jnp.where(_eyesel(), v01, 0.0), axis=1, keepdims=True)


def _col_to_row(v01):
    return jnp.sum(jnp.where(_eyesel(), v01, 0.0), axis=0, keepdims=True)


def _iou_gt(bx1, by1, bx2, by2, a1, tx1, ty1, tx2, ty2, a2):
    ix1 = jnp.maximum(bx1, tx1)
    iy1 = jnp.maximum(by1, ty1)
    ix2 = jnp.minimum(bx2, tx2)
    iy2 = jnp.minimum(by2, ty2)
    inter = jnp.clip(ix2 - ix1, 0.0) * jnp.clip(iy2 - iy1, 0.0)
    return (inter / (a1 + a2 - inter + 1e-9)) > IOU


def _nms_kernel(xt_ref, out_ref, dt_ref, st_ref, rkc_ref, krow_ref,
                mu_ref, ml_ref, hs_ref, eye_ref, sur_ref, nk_ref):
    f32 = F32
    eye_ref[...] = jnp.where(_iotai((B, B), 0) == _iotai((B, B), 1),
                             1.0, 0.0).astype(BF16)
    eyeb = eye_ref[...]

    xr = xt_ref[...]
    cxr = xr[0:1, :] * 640.0
    cyr = xr[1:2, :] * 640.0
    wr = xr[2:3, :] * 640.0
    hr = xr[3:4, :] * 640.0
    sr = xr[4:5, :] * xr[5:6, :]
    smr = jnp.where(sr > CONF, sr, -1.0)
    dtv = jnp.concatenate(
        [cxr - wr / 2, cyr - hr / 2, cxr + wr / 2, cyr + hr / 2, smr,
         jnp.zeros((3, N), f32)], axis=0)
    dt_ref[...] = dtv

    h1, h2, h3 = _split3(dtv)
    hs_ref[...] = jnp.concatenate([h1, h2, h3], axis=0)

    tri = _iotai((B, 1), 0) > _iotai((1, B), 1)
    acc_rows = [jnp.zeros((1, B), f32) for _ in range(NB)]
    for t in range(NB):
        ts = slice(t * B, (t + 1) * B)
        sit = dt_ref[4:5, ts]
        si = _row_to_col(sit)
        diag = (sit > si) | ((sit == si) & tri)
        acc = jnp.sum(diag.astype(f32), axis=1, keepdims=True)
        for j in range(t + 1, NB):
            sj = dt_ref[4:5, j * B:(j + 1) * B]
            a = (sj > si).astype(f32)
            acc = acc + jnp.sum(a, axis=1, keepdims=True)
            acc_rows[j] = acc_rows[j] + (
                float(B) - jnp.sum(a, axis=0, keepdims=True))
        rkc_ref[:, t:t + 1] = acc + _row_to_col(acc_rows[t])

    dn = (((1,), (0,)), ((), ()))
    nvalid = jnp.sum((smr > CONF).astype(f32))
    for t in range(NB):
        ts = slice(t * B, (t + 1) * B)

        @pl.when(nvalid > float(t * B))
        def _gather(ts=ts, t=t):
            kkr = _iota((1, B), 1) + t * B
            acc = jnp.zeros((24, B), f32)
            for j in range(NB):
                js = slice(j * B, (j + 1) * B)
                q = (rkc_ref[:, j:j + 1] == kkr).astype(BF16)
                acc = acc + jax.lax.dot_general(
                    hs_ref[:, js], q, dn, preferred_element_type=f32)
            st_ref[:, ts] = acc[0:8] + (acc[8:16] + acc[16:24])

        @pl.when(nvalid <= float(t * B))
        def _zero(ts=ts):
            st_ref[:, ts] = jnp.zeros((8, B), f32)

    krow_ref[...] = (st_ref[4:5, :] > CONF).astype(f32)
    ar = (jnp.clip(st_ref[2:3, :] - st_ref[0:1, :], 0.0) *
          jnp.clip(st_ref[3:4, :] - st_ref[1:2, :], 0.0))

    one = jnp.ones((B, B), f32)
    zero = jnp.zeros((B, B), f32)
    ri = _iotai((B, B), 0)
    ci = _iotai((B, B), 1)
    W = 64
    BPB = B // W
    for b in range(NB):
        nk_ref[0, b] = 0.0

    for c in range(NB):
        cs = slice(c * B, (c + 1) * B)

        @pl.when(jnp.sum(krow_ref[0:1, cs]) > 0.0)
        def _block(c=c, cs=cs):
            tx1 = st_ref[0:1, cs]
            ty1 = st_ref[1:2, cs]
            tx2 = st_ref[2:3, cs]
            ty2 = st_ref[3:4, cs]
            arc = ar[0:1, cs]

            def band(idx, _):
                b2 = idx // BPB
                m = idx - b2 * BPB

                @pl.when(nk_ref[0, b2] > (m * W).astype(f32))
                def _band():
                    r0 = b2 * B + m * W
                    sx1 = sur_ref[pl.ds(r0, W), 0:1]
                    sy1 = sur_ref[pl.ds(r0, W), 1:2]
                    sx2 = sur_ref[pl.ds(r0, W), 2:3]
                    sy2 = sur_ref[pl.ds(r0, W), 3:4]
                    sa = sur_ref[pl.ds(r0, W), 4:5]
                    gtb = _iou_gt(sx1, sy1, sx2, sy2, sa,
                                  tx1, ty1, tx2, ty2, arc)
                    supb = jnp.max(jnp.where(gtb, 1.0, 0.0),
                                   axis=0, keepdims=True)
                    ka = krow_ref[0:1, cs]
                    krow_ref[0:1, cs] = ka * (1.0 - supb)
                return 0

            if c > 0:
                jax.lax.fori_loop(0, c * BPB, band, 0)
            v_row = krow_ref[0:1, cs]

            @pl.when(jnp.sum(v_row) > 0.0)
            def _resolve():
                p1, p2, p3 = _split3(st_ref[:, cs])
                tb = _t_pieces(eyeb, p1, p2, p3)
                bx1 = tb[:, 0:1]
                by1 = tb[:, 1:2]
                bx2 = tb[:, 2:3]
                by2 = tb[:, 3:4]
                a1 = (jnp.clip(bx2 - bx1, 0.0) *
                      jnp.clip(by2 - by1, 0.0))
                gt = _iou_gt(bx1, by1, bx2, by2, a1,
                             tx1, ty1, tx2, ty2, arc)
                mu_ref[...] = jnp.where(gt & (ri < ci), one, zero)
                ml_ref[...] = jnp.where(gt & (ri > ci), one, zero)
                v_col = _row_to_col(v_row)

                def cond(st):
                    _, t, diff = st
                    return (diff > 0.0) & (t < B + 2)

                def body(st):
                    kc, t, _ = st
                    sup_r = jnp.max(mu_ref[...] * kc, axis=0,
                                    keepdims=True)
                    kr = v_row * (1.0 - sup_r)
                    sup_c = jnp.max(ml_ref[...] * kr, axis=1,
                                    keepdims=True)
                    kc2 = v_col * (1.0 - sup_c)
                    return kc2, t + 1, jnp.sum(jnp.abs(kc2 - kc))

                kc_fin, _, _ = jax.lax.while_loop(
                    cond, body, (v_col, jnp.int32(0), jnp.float32(1.0)))
                kr_fin = _col_to_row(kc_fin)
                krow_ref[0:1, cs] = kr_fin

                prefix = jnp.sum(jnp.where(ri > ci, kr_fin, 0.0),
                                 axis=1, keepdims=True)
                kk = _iota((1, B), 1)
                qc = jnp.where((prefix == kk) & (kc_fin > 0.0),
                               1.0, 0.0).astype(BF16)
                hsb = jnp.concatenate([p1, p2, p3], axis=0)
                cp = jax.lax.dot_general(hsb, qc, dn,
                                         preferred_element_type=f32)
                cc = _t_pieces(eyeb, cp[0:8].astype(BF16),
                               cp[8:16].astype(BF16),
                               cp[16:24].astype(BF16))
                sa1 = (jnp.clip(cc[:, 2:3] - cc[:, 0:1], 0.0) *
                       jnp.clip(cc[:, 3:4] - cc[:, 1:2], 0.0))
                sur_ref[cs, 0:4] = cc[:, 0:4]
                sur_ref[cs, 4:5] = sa1
                nk_ref[0, c] = jnp.sum(kc_fin)

    out_ref[...] = st_ref[...] * krow_ref[...]


@jax.jit
def kernel(x):
    p = x[0]
    xt = jnp.zeros((8, N), F32).at[:6, :N_RAW].set(p.T)
    out_t = pl.pallas_call(
        _nms_kernel,
        out_shape=jax.ShapeDtypeStruct((8, N), F32),
        scratch_shapes=[
            pltpu.VMEM((8, N), F32),
            pltpu.VMEM((8, N), F32),
            pltpu.VMEM((B, 16), F32),
            pltpu.VMEM((1, N), F32),
            pltpu.VMEM((B, B), F32),
            pltpu.VMEM((B, B), F32),
            pltpu.VMEM((24, N), BF16),
            pltpu.VMEM((B, B), BF16),
            pltpu.VMEM((N, 8), F32),
            pltpu.SMEM((1, 16), F32),
        ],
    )(xt)
    return out_t[:5, :N_RAW].T


# speedup vs baseline: 1.6545x; 1.0156x over previous
import jax
import jax.numpy as jnp
from jax.experimental import pallas as pl
from jax.experimental.pallas import tpu as pltpu

N_RAW = 5000
N = 5120
B = 512
NB = N // B
CONF = 0.25
IOU = 0.45
F32 = jnp.float32
BF16 = jnp.bfloat16


def _iotai(shape, dim):
    return jax.lax.broadcasted_iota(jnp.int32, shape, dim)


def _iota(shape, dim):
    return _iotai(shape, dim).astype(F32)


def _split3(v):
    p1 = v.astype(BF16)
    r = v - p1.astype(F32)
    p2 = r.astype(BF16)
    p3 = (r - p2.astype(F32)).astype(BF16)
    return p1, p2, p3


_DN_T = (((1,), (1,)), ((), ()))


def _t_pieces(eyeb, p1, p2, p3):
    d = lambda p: jax.lax.dot_general(eyeb, p, _DN_T,
                                      preferred_element_type=F32)
    return d(p1) + (d(p2) + d(p3))


def _t_block(eyeb, blk):
    return _t_pieces(eyeb, *_split3(blk))


def _eyesel():
    return _iotai((B, B), 0) == _iotai((B, B), 1)


def _row_to_col(v01):
    return jnp.sum(jnp.where(_eyesel(), v01, 0.0), axis=1, keepdims=True)


def _col_to_row(v01):
    return jnp.sum(jnp.where(_eyesel(), v01, 0.0), axis=0, keepdims=True)


def _iou_gt(bx1, by1, bx2, by2, a1, tx1, ty1, tx2, ty2, a2):
    ix1 = jnp.maximum(bx1, tx1)
    iy1 = jnp.maximum(by1, ty1)
    ix2 = jnp.minimum(bx2, tx2)
    iy2 = jnp.minimum(by2, ty2)
    inter = jnp.clip(ix2 - ix1, 0.0) * jnp.clip(iy2 - iy1, 0.0)
    return (inter / (a1 + a2 - inter + 1e-9)) > IOU


def _nms_kernel(xt_ref, out_ref, dt_ref, st_ref, rkc_ref, krow_ref,
                mu_ref, ml_ref, hs_ref, eye_ref, sur_ref, nk_ref):
    f32 = F32
    eye_ref[...] = jnp.where(_iotai((B, B), 0) == _iotai((B, B), 1),
                             1.0, 0.0).astype(BF16)
    eyeb = eye_ref[...]

    xr = xt_ref[...]
    cxr = xr[0:1, :] * 640.0
    cyr = xr[1:2, :] * 640.0
    wr = xr[2:3, :] * 640.0
    hr = xr[3:4, :] * 640.0
    sr = xr[4:5, :] * xr[5:6, :]
    smr = jnp.where(sr > CONF, sr, -1.0)
    dtv = jnp.concatenate(
        [cxr - wr / 2, cyr - hr / 2, cxr + wr / 2, cyr + hr / 2, smr,
         jnp.zeros((3, N), f32)], axis=0)
    dt_ref[...] = dtv

    h1, h2, h3 = _split3(dtv)
    hs_ref[...] = jnp.concatenate([h1, h2, h3], axis=0)

    tri = _iotai((B, 1), 0) > _iotai((1, B), 1)
    acc_rows = [jnp.zeros((1, B), f32) for _ in range(NB)]
    for t in range(NB):
        ts = slice(t * B, (t + 1) * B)
        sit = dt_ref[4:5, ts]
        si = _row_to_col(sit)
        diag = (sit > si) | ((sit == si) & tri)
        acc = jnp.sum(diag.astype(f32), axis=1, keepdims=True)
        for j in range(t + 1, NB):
            sj = dt_ref[4:5, j * B:(j + 1) * B]
            a = (sj > si).astype(f32)
            acc = acc + jnp.sum(a, axis=1, keepdims=True)
            acc_rows[j] = acc_rows[j] + (
                float(B) - jnp.sum(a, axis=0, keepdims=True))
        rkc_ref[:, t:t + 1] = acc + _row_to_col(acc_rows[t])

    dn = (((1,), (0,)), ((), ()))
    nvalid = jnp.sum((smr > CONF).astype(f32))
    for t in range(NB):
        ts = slice(t * B, (t + 1) * B)

        @pl.when(nvalid > float(t * B))
        def _gather(ts=ts, t=t):
            kkr = _iota((1, B), 1) + t * B
            acc = jnp.zeros((24, B), f32)
            for j in range(NB):
                js = slice(j * B, (j + 1) * B)
                q = (rkc_ref[:, j:j + 1] == kkr).astype(BF16)
                acc = acc + jax.lax.dot_general(
                    hs_ref[:, js], q, dn, preferred_element_type=f32)
            st_ref[:, ts] = acc[0:8] + (acc[8:16] + acc[16:24])

        @pl.when(nvalid <= float(t * B))
        def _zero(ts=ts):
            st_ref[:, ts] = jnp.zeros((8, B), f32)

    krow_ref[...] = (st_ref[4:5, :] > CONF).astype(f32)
    ar = (jnp.clip(st_ref[2:3, :] - st_ref[0:1, :], 0.0) *
          jnp.clip(st_ref[3:4, :] - st_ref[1:2, :], 0.0))

    one = jnp.ones((B, B), f32)
    zero = jnp.zeros((B, B), f32)
    ri = _iotai((B, B), 0)
    ci = _iotai((B, B), 1)
    W = 128
    BPB = B // W
    for b in range(NB):
        nk_ref[0, b] = 0.0

    for c in range(NB):
        cs = slice(c * B, (c + 1) * B)

        @pl.when(jnp.sum(krow_ref[0:1, cs]) > 0.0)
        def _block(c=c, cs=cs):
            tx1 = st_ref[0:1, cs]
            ty1 = st_ref[1:2, cs]
            tx2 = st_ref[2:3, cs]
            ty2 = st_ref[3:4, cs]
            arc = ar[0:1, cs]

            def band(idx, _):
                b2 = idx // BPB
                m = idx - b2 * BPB

                @pl.when(nk_ref[0, b2] > (m * W).astype(f32))
                def _band():
                    r0 = b2 * B + m * W
                    sx1 = sur_ref[pl.ds(r0, W), 0:1]
                    sy1 = sur_ref[pl.ds(r0, W), 1:2]
                    sx2 = sur_ref[pl.ds(r0, W), 2:3]
                    sy2 = sur_ref[pl.ds(r0, W), 3:4]
                    sa = sur_ref[pl.ds(r0, W), 4:5]
                    gtb = _iou_gt(sx1, sy1, sx2, sy2, sa,
                                  tx1, ty1, tx2, ty2, arc)
                    supb = jnp.max(jnp.where(gtb, 1.0, 0.0),
                                   axis=0, keepdims=True)
                    ka = krow_ref[0:1, cs]
                    krow_ref[0:1, cs] = ka * (1.0 - supb)
                return 0

            if c > 0:
                jax.lax.fori_loop(0, c * BPB, band, 0)
            v_row = krow_ref[0:1, cs]

            @pl.when(jnp.sum(v_row) > 0.0)
            def _resolve():
                p1, p2, p3 = _split3(st_ref[:, cs])
                tb = _t_pieces(eyeb, p1, p2, p3)
                bx1 = tb[:, 0:1]
                by1 = tb[:, 1:2]
                bx2 = tb[:, 2:3]
                by2 = tb[:, 3:4]
                a1 = (jnp.clip(bx2 - bx1, 0.0) *
                      jnp.clip(by2 - by1, 0.0))
                gt = _iou_gt(bx1, by1, bx2, by2, a1,
                             tx1, ty1, tx2, ty2, arc)
                mu_ref[...] = jnp.where(gt & (ri < ci), one, zero)
                ml_ref[...] = jnp.where(gt & (ri > ci), one, zero)
                v_col = _row_to_col(v_row)

                def cond(st):
                    _, t, diff = st
                    return (diff > 0.0) & (t < B + 2)

                def body(st):
                    kc, t, _ = st
                    sup_r = jnp.max(mu_ref[...] * kc, axis=0,
                                    keepdims=True)
                    kr = v_row * (1.0 - sup_r)
                    sup_c = jnp.max(ml_ref[...] * kr, axis=1,
                                    keepdims=True)
                    kc2 = v_col * (1.0 - sup_c)
                    return kc2, t + 1, jnp.sum(jnp.abs(kc2 - kc))

                kc_fin, _, _ = jax.lax.while_loop(
                    cond, body, (v_col, jnp.int32(0), jnp.float32(1.0)))
                kr_fin = _col_to_row(kc_fin)
                krow_ref[0:1, cs] = kr_fin

                prefix = jnp.sum(jnp.where(ri > ci, kr_fin, 0.0),
                                 axis=1, keepdims=True)
                kk = _iota((1, B), 1)
                qc = jnp.where((prefix == kk) & (kc_fin > 0.0),
                               1.0, 0.0).astype(BF16)
                hsb = jnp.concatenate([p1, p2, p3], axis=0)
                cp = jax.lax.dot_general(hsb, qc, dn,
                                         preferred_element_type=f32)
                cc = _t_pieces(eyeb, cp[0:8].astype(BF16),
                               cp[8:16].astype(BF16),
                               cp[16:24].astype(BF16))
                sa1 = (jnp.clip(cc[:, 2:3] - cc[:, 0:1], 0.0) *
                       jnp.clip(cc[:, 3:4] - cc[:, 1:2], 0.0))
                sur_ref[cs, 0:4] = cc[:, 0:4]
                sur_ref[cs, 4:5] = sa1
                nk_ref[0, c] = jnp.sum(kc_fin)

    out_ref[...] = st_ref[...] * krow_ref[...]


@jax.jit
def kernel(x):
    p = x[0]
    xt = jnp.zeros((8, N), F32).at[:6, :N_RAW].set(p.T)
    out_t = pl.pallas_call(
        _nms_kernel,
        out_shape=jax.ShapeDtypeStruct((8, N), F32),
        scratch_shapes=[
            pltpu.VMEM((8, N), F32),
            pltpu.VMEM((8, N), F32),
            pltpu.VMEM((B, 16), F32),
            pltpu.VMEM((1, N), F32),
            pltpu.VMEM((B, B), F32),
            pltpu.VMEM((B, B), F32),
            pltpu.VMEM((24, N), BF16),
            pltpu.VMEM((B, B), BF16),
            pltpu.VMEM((N, 8), F32),
            pltpu.SMEM((1, 16), F32),
        ],
    )(xt)
    return out_t[:5, :N_RAW].T


# speedup vs baseline: 1.6545x; 1.0000x over previous
import jax
import jax.numpy as jnp
from jax.experimental import pallas as pl
from jax.experimental.pallas import tpu as pltpu

N_RAW = 5000
N = 5120
B = 512
NB = N // B
CONF = 0.25
IOU = 0.45
F32 = jnp.float32
BF16 = jnp.bfloat16


def _iotai(shape, dim):
    return jax.lax.broadcasted_iota(jnp.int32, shape, dim)


def _iota(shape, dim):
    return _iotai(shape, dim).astype(F32)


def _split3(v):
    p1 = v.astype(BF16)
    r = v - p1.astype(F32)
    p2 = r.astype(BF16)
    p3 = (r - p2.astype(F32)).astype(BF16)
    return p1, p2, p3


_DN_T = (((1,), (1,)), ((), ()))


def _t_pieces(eyeb, p1, p2, p3):
    d = lambda p: jax.lax.dot_general(eyeb, p, _DN_T,
                                      preferred_element_type=F32)
    return d(p1) + (d(p2) + d(p3))


def _eyesel():
    return _iotai((B, B), 0) == _iotai((B, B), 1)


def _row_to_col(v01):
    return jnp.sum(jnp.where(_eyesel(), v01, 0.0), axis=1, keepdims=True)


def _col_to_row(v01):
    return jnp.sum(jnp.where(_eyesel(), v01, 0.0), axis=0, keepdims=True)


def _iou_gt(bx1, by1, bx2, by2, a1, tx1, ty1, tx2, ty2, a2):
    ix1 = jnp.maximum(bx1, tx1)
    iy1 = jnp.maximum(by1, ty1)
    ix2 = jnp.minimum(bx2, tx2)
    iy2 = jnp.minimum(by2, ty2)
    inter = jnp.clip(ix2 - ix1, 0.0) * jnp.clip(iy2 - iy1, 0.0)
    return (inter / (a1 + a2 - inter + 1e-9)) > IOU


def _nms_kernel(xt_ref, out_ref, dt_ref, st_ref, rkc_ref, krow_ref,
                mu_ref, ml_ref, hs_ref, eye_ref, sur_ref, nk_ref):
    f32 = F32
    eye_ref[...] = jnp.where(_iotai((B, B), 0) == _iotai((B, B), 1),
                             1.0, 0.0).astype(BF16)
    eyeb = eye_ref[...]

    xr = xt_ref[...]
    cxr = xr[0:1, :] * 640.0
    cyr = xr[1:2, :] * 640.0
    wr = xr[2:3, :] * 640.0
    hr = xr[3:4, :] * 640.0
    sr = xr[4:5, :] * xr[5:6, :]
    smr = jnp.where(sr > CONF, sr, -1.0)
    dtv = jnp.concatenate(
        [cxr - wr / 2, cyr - hr / 2, cxr + wr / 2, cyr + hr / 2, smr,
         jnp.zeros((3, N), f32)], axis=0)
    dt_ref[...] = dtv

    h1, h2, h3 = _split3(dtv)
    hs_ref[...] = jnp.concatenate([h1, h2, h3], axis=0)

    tri = _iotai((B, 1), 0) > _iotai((1, B), 1)
    acc_rows = [jnp.zeros((1, B), f32) for _ in range(NB)]
    for t in range(NB):
        ts = slice(t * B, (t + 1) * B)
        sit = dt_ref[4:5, ts]
        si = _row_to_col(sit)
        diag = (sit > si) | ((sit == si) & tri)
        acc = jnp.sum(diag.astype(f32), axis=1, keepdims=True)
        for j in range(t + 1, NB):
            sj = dt_ref[4:5, j * B:(j + 1) * B]
            a = (sj > si).astype(f32)
            acc = acc + jnp.sum(a, axis=1, keepdims=True)
            acc_rows[j] = acc_rows[j] + (
                float(B) - jnp.sum(a, axis=0, keepdims=True))
        rkc_ref[:, t:t + 1] = acc + _row_to_col(acc_rows[t])

    dn = (((1,), (0,)), ((), ()))
    nvalid = jnp.sum((smr > CONF).astype(f32))
    for t in range(NB):
        ts = slice(t * B, (t + 1) * B)

        @pl.when(nvalid > float(t * B))
        def _gather(ts=ts, t=t):
            kkr = _iota((1, B), 1) + t * B
            acc = jnp.zeros((24, B), f32)
            for j in range(NB):
                js = slice(j * B, (j + 1) * B)
                q = (rkc_ref[:, j:j + 1] == kkr).astype(BF16)
                acc = acc + jax.lax.dot_general(
                    hs_ref[:, js], q, dn, preferred_element_type=f32)
            st_ref[:, ts] = acc[0:8] + (acc[8:16] + acc[16:24])

        @pl.when(nvalid <= float(t * B))
        def _zero(ts=ts):
            st_ref[:, ts] = jnp.zeros((8, B), f32)

    krow_ref[...] = (st_ref[4:5, :] > CONF).astype(f32)
    ar = (jnp.clip(st_ref[2:3, :] - st_ref[0:1, :], 0.0) *
          jnp.clip(st_ref[3:4, :] - st_ref[1:2, :], 0.0))

    one = jnp.ones((B, B), f32)
    zero = jnp.zeros((B, B), f32)
    ri = _iotai((B, B), 0)
    ci = _iotai((B, B), 1)
    W = 128
    BPB = B // W
    for b in range(NB):
        nk_ref[0, b] = 0.0

    for c in range(NB):
        cs = slice(c * B, (c + 1) * B)

        @pl.when(jnp.sum(krow_ref[0:1, cs]) > 0.0)
        def _block(c=c, cs=cs):
            tx1 = st_ref[0:1, cs]
            ty1 = st_ref[1:2, cs]
            tx2 = st_ref[2:3, cs]
            ty2 = st_ref[3:4, cs]
            arc = ar[0:1, cs]

            def band(idx, _):
                b2 = idx // BPB
                m = idx - b2 * BPB

                @pl.when(nk_ref[0, b2] > (m * W).astype(f32))
                def _band():
                    r0 = b2 * B + m * W
                    sx1 = sur_ref[pl.ds(r0, W), 0:1]
                    sy1 = sur_ref[pl.ds(r0, W), 1:2]
                    sx2 = sur_ref[pl.ds(r0, W), 2:3]
                    sy2 = sur_ref[pl.ds(r0, W), 3:4]
                    sa = sur_ref[pl.ds(r0, W), 4:5]
                    gtb = _iou_gt(sx1, sy1, sx2, sy2, sa,
                                  tx1, ty1, tx2, ty2, arc)
                    supb = jnp.max(jnp.where(gtb, 1.0, 0.0),
                                   axis=0, keepdims=True)
                    ka = krow_ref[0:1, cs]
                    krow_ref[0:1, cs] = ka * (1.0 - supb)
                return 0

            if c > 0:
                jax.lax.fori_loop(0, c * BPB, band, 0)
            v_row = krow_ref[0:1, cs]

            @pl.when(jnp.sum(v_row) > 0.0)
            def _resolve():
                p1, p2, p3 = _split3(st_ref[:, cs])
                tb = _t_pieces(eyeb, p1, p2, p3)
                bx1 = tb[:, 0:1]
                by1 = tb[:, 1:2]
                bx2 = tb[:, 2:3]
                by2 = tb[:, 3:4]
                a1 = (jnp.clip(bx2 - bx1, 0.0) *
                      jnp.clip(by2 - by1, 0.0))
                gt = _iou_gt(bx1, by1, bx2, by2, a1,
                             tx1, ty1, tx2, ty2, arc)
                mu_ref[...] = jnp.where(gt & (ri < ci), one, zero)
                ml_ref[...] = jnp.where(gt & (ri > ci), one, zero)
                v_col = _row_to_col(v_row)

                def cond(st):
                    _, t, diff = st
                    return (diff > 0.0) & (t < B + 2)

                def body(st):
                    kc, t, _ = st
                    sup_r = jnp.max(mu_ref[...] * kc, axis=0,
                                    keepdims=True)
                    kr = v_row * (1.0 - sup_r)
                    sup_c = jnp.max(ml_ref[...] * kr, axis=1,
                                    keepdims=True)
                    kc2 = v_col * (1.0 - sup_c)
                    return kc2, t + 1, jnp.sum(jnp.abs(kc2 - kc))

                kc_fin, _, _ = jax.lax.while_loop(
                    cond, body, (v_col, jnp.int32(0), jnp.float32(1.0)))
                kr_fin = _col_to_row(kc_fin)
                krow_ref[0:1, cs] = kr_fin

                prefix = jnp.sum(jnp.where(ri > ci, kr_fin, 0.0),
                                 axis=1, keepdims=True)
                kk = _iota((1, B), 1)
                qc = jnp.where((prefix == kk) & (kc_fin > 0.0),
                               1.0, 0.0).astype(BF16)
                hsb = jnp.concatenate([p1, p2, p3], axis=0)
                cp = jax.lax.dot_general(hsb, qc, dn,
                                         preferred_element_type=f32)
                cc = _t_pieces(eyeb, cp[0:8].astype(BF16),
                               cp[8:16].astype(BF16),
                               cp[16:24].astype(BF16))
                sa1 = (jnp.clip(cc[:, 2:3] - cc[:, 0:1], 0.0) *
                       jnp.clip(cc[:, 3:4] - cc[:, 1:2], 0.0))
                sur_ref[cs, 0:4] = cc[:, 0:4]
                sur_ref[cs, 4:5] = sa1
                nk_ref[0, c] = jnp.sum(kc_fin)

    out_ref[...] = st_ref[...] * krow_ref[...]


@jax.jit
def kernel(x):
    p = x[0]
    xt = jnp.zeros((8, N), F32).at[:6, :N_RAW].set(p.T)
    out_t = pl.pallas_call(
        _nms_kernel,
        out_shape=jax.ShapeDtypeStruct((8, N), F32),
        scratch_shapes=[
            pltpu.VMEM((8, N), F32),
            pltpu.VMEM((8, N), F32),
            pltpu.VMEM((B, 16), F32),
            pltpu.VMEM((1, N), F32),
            pltpu.VMEM((B, B), F32),
            pltpu.VMEM((B, B), F32),
            pltpu.VMEM((24, N), BF16),
            pltpu.VMEM((B, B), BF16),
            pltpu.VMEM((N, 8), F32),
            pltpu.SMEM((1, 16), F32),
        ],
    )(xt)
    return out_t[:5, :N_RAW].T
